# baseline (speedup 1.0000x reference)
import jax
import jax.numpy as jnp
import numpy as np
from jax import lax
from jax.experimental import pallas as pl
from jax.experimental.pallas import tpu as pltpu
from jax.experimental.pallas import tpu_sc as plsc

N_LEVELS = 12
F_PER = 2
LOG2_T = 19
T = 1 << LOG2_T
BASE_RES = 16
PER_LEVEL_SCALE = 2.0
N_POINTS = 524288

P1 = int(np.uint32(2654435761).view(np.int32))
P2 = int(np.uint32(805459861).view(np.int32))

NC, NS = 2, 16
NW = NC * NS
P_PER_W = N_POINTS // NW
C = 128
N_CHUNKS = P_PER_W // C
N_ROWS = N_LEVELS * 8 * 2

_RES = [int(np.ceil(BASE_RES * (PER_LEVEL_SCALE ** l))) for l in range(N_LEVELS)]
_DENSE = [(r + 1) ** 3 <= T for r in _RES]


def _fire_level(tbl_hbm, idx_v, feat_v, sem, l):
    for r in range(l * 16, l * 16 + 16):
        pltpu.make_async_copy(
            tbl_hbm.at[idx_v.at[r]], feat_v.at[pl.ds(r * C, C)], sem
        ).start()


def _drain_level(tbl_hbm, idx_v, feat_v, sem, l):
    for r in range(l * 16, l * 16 + 16):
        pltpu.make_async_copy(
            tbl_hbm.at[idx_v.at[r]], feat_v.at[pl.ds(r * C, C)], sem
        ).wait()


def _tec_body(x_hbm, tbl_hbm, out_hbm, x_v, idx_v, feat_v, out_v, sem):
    wid = lax.axis_index("s") * NC + lax.axis_index("c")

    def chunk_body(ci, _):
        base = wid * P_PER_W + ci * C
        pltpu.sync_copy(x_hbm.at[:, pl.ds(base, C)], x_v)

        for l in range(N_LEVELS):
            res = _RES[l]
            lT2 = 2 * l * T

            def idx_g(g, _, l=l, res=res, lT2=lT2):
                s16 = pl.ds(g * 16, 16)
                px = x_v[0, s16] * jnp.float32(res)
                py = x_v[1, s16] * jnp.float32(res)
                pz = x_v[2, s16] * jnp.float32(res)
                ix = px.astype(jnp.int32)
                iy = py.astype(jnp.int32)
                iz = pz.astype(jnp.int32)
                if _DENSE[l]:
                    s = res + 1
                    y0 = iy * jnp.int32(2 * s)
                    y1 = y0 + jnp.int32(2 * s)
                    z0 = iz * jnp.int32(2 * s * s)
                    z1 = z0 + jnp.int32(2 * s * s)
                    x0 = ix + ix + jnp.int32(lT2)
                    x1 = x0 + jnp.int32(2)
                    for c in range(8):
                        e = ((x1 if (c & 1) else x0)
                             + (y1 if (c & 2) else y0)
                             + (z1 if (c & 4) else z0))
                        r = (l * 8 + c) * 2
                        idx_v[r, s16] = e
                        idx_v[r + 1, s16] = e + jnp.int32(1)
                else:
                    y0 = iy * jnp.int32(P1)
                    y1 = y0 + jnp.int32(P1)
                    z0 = iz * jnp.int32(P2)
                    z1 = z0 + jnp.int32(P2)
                    x1 = ix + jnp.int32(1)
                    for c in range(8):
                        h = ((x1 if (c & 1) else ix)
                             ^ (y1 if (c & 2) else y0)
                             ^ (z1 if (c & 4) else z0)) & jnp.int32(T - 1)
                        e = h + h + jnp.int32(lT2)
                        r = (l * 8 + c) * 2
                        idx_v[r, s16] = e
                        idx_v[r + 1, s16] = e + jnp.int32(1)
                return 0

            lax.fori_loop(0, C // 16, idx_g, 0, unroll=False)
            _fire_level(tbl_hbm, idx_v, feat_v, sem, l)
            if l >= 2:
                _drain_level(tbl_hbm, idx_v, feat_v, sem, l - 2)

        _drain_level(tbl_hbm, idx_v, feat_v, sem, N_LEVELS - 2)
        _drain_level(tbl_hbm, idx_v, feat_v, sem, N_LEVELS - 1)

        for l in range(N_LEVELS):
            res = _RES[l]

            def acc_g(g, _, l=l, res=res):
                s16 = pl.ds(g * 16, 16)
                px = x_v[0, s16] * jnp.float32(res)
                py = x_v[1, s16] * jnp.float32(res)
                pz = x_v[2, s16] * jnp.float32(res)
                wx = px - px.astype(jnp.int32).astype(jnp.float32)
                wy = py - py.astype(jnp.int32).astype(jnp.float32)
                wz = pz - pz.astype(jnp.int32).astype(jnp.float32)
                ox = jnp.float32(1.0) - wx
                oy = jnp.float32(1.0) - wy
                oz = jnp.float32(1.0) - wz
                acc0 = jnp.zeros((16,), jnp.float32)
                acc1 = jnp.zeros((16,), jnp.float32)
                for c in range(8):
                    sx = wx if (c & 1) else ox
                    sy = wy if (c & 2) else oy
                    sz = wz if (c & 4) else oz
                    wc = (sx * sy) * sz
                    r = (l * 8 + c) * 2
                    f0 = feat_v[pl.ds(r * C + g * 16, 16)]
                    f1 = feat_v[pl.ds((r + 1) * C + g * 16, 16)]
                    acc0 = acc0 + f0 * wc
                    acc1 = acc1 + f1 * wc
                out_v[pl.ds((2 * l) * C + g * 16, 16)] = acc0
                out_v[pl.ds((2 * l + 1) * C + g * 16, 16)] = acc1
                return 0

            lax.fori_loop(0, C // 16, acc_g, 0, unroll=False)

        for p in range(N_LEVELS * F_PER):
            pltpu.sync_copy(
                out_v.at[pl.ds(p * C, C)], out_hbm.at[p, pl.ds(base, C)]
            )
        return 0

    lax.fori_loop(0, N_CHUNKS, chunk_body, 0, unroll=False)


@jax.jit
def kernel(x, table, mask):
    del mask
    xT = jnp.transpose(x)
    tbl = table.reshape(N_LEVELS * T * F_PER)

    mesh = plsc.VectorSubcoreMesh(
        core_axis_name="c", subcore_axis_name="s", num_cores=NC, num_subcores=NS
    )
    f = pl.kernel(
        _tec_body,
        out_type=jax.ShapeDtypeStruct((N_LEVELS * F_PER, N_POINTS), jnp.float32),
        mesh=mesh,
        scratch_types=[
            pltpu.VMEM((3, C), jnp.float32),
            pltpu.VMEM((N_ROWS, C), jnp.int32),
            pltpu.VMEM((N_ROWS * C,), jnp.float32),
            pltpu.VMEM((N_LEVELS * F_PER * C,), jnp.float32),
            pltpu.SemaphoreType.DMA,
        ],
    )
    planes = f(xT, tbl)
    return planes.reshape(N_LEVELS, F_PER, N_POINTS).transpose(2, 0, 1).reshape(
        N_POINTS, N_LEVELS * F_PER
    )


# speedup vs baseline: 2.6635x; 2.6635x over previous
import jax
import jax.numpy as jnp
import numpy as np
from jax import lax
from jax.experimental import pallas as pl
from jax.experimental.pallas import tpu as pltpu
from jax.experimental.pallas import tpu_sc as plsc

N_LEVELS = 12
F_PER = 2
LOG2_T = 19
T = 1 << LOG2_T
BASE_RES = 16
PER_LEVEL_SCALE = 2.0
N_POINTS = 524288

P1 = int(np.uint32(2654435761).view(np.int32))
P2 = int(np.uint32(805459861).view(np.int32))

NC, NS = 2, 16
NW = NC * NS
P_PER_W = N_POINTS // NW
C = 128
N_CHUNKS = P_PER_W // C
E_PER_F = N_LEVELS * 8 * C
NF = N_LEVELS * F_PER
E_OUT = NF * C

_RES = [int(np.ceil(BASE_RES * (PER_LEVEL_SCALE ** l))) for l in range(N_LEVELS)]
_DENSE = [(r + 1) ** 3 <= T for r in _RES]


def _index_pass(x_v, i0_v, i1_v):
    for l in range(N_LEVELS):
        res = _RES[l]
        lT2 = l * T

        def idx_g(g, _, l=l, res=res, lT2=lT2):
            px = x_v[pl.ds(0 * C + g * 16, 16)] * jnp.float32(res)
            py = x_v[pl.ds(1 * C + g * 16, 16)] * jnp.float32(res)
            pz = x_v[pl.ds(2 * C + g * 16, 16)] * jnp.float32(res)
            ix = px.astype(jnp.int32)
            iy = py.astype(jnp.int32)
            iz = pz.astype(jnp.int32)
            if _DENSE[l]:
                s = res + 1
                y0 = iy * jnp.int32(s)
                y1 = y0 + jnp.int32(s)
                z0 = iz * jnp.int32(s * s)
                z1 = z0 + jnp.int32(s * s)
                x0 = ix + jnp.int32(lT2)
                x1 = x0 + jnp.int32(1)
                for c in range(8):
                    h = ((x1 if (c & 1) else x0)
                         + (y1 if (c & 2) else y0)
                         + (z1 if (c & 4) else z0))
                    e = h + (h & jnp.int32(-128))
                    off = (l * 8 + c) * C
                    i0_v[pl.ds(off + g * 16, 16)] = e
                    i1_v[pl.ds(off + g * 16, 16)] = e + jnp.int32(128)
            else:
                y0 = iy * jnp.int32(P1)
                y1 = y0 + jnp.int32(P1)
                z0 = iz * jnp.int32(P2)
                z1 = z0 + jnp.int32(P2)
                x1 = ix + jnp.int32(1)
                for c in range(8):
                    h = (((x1 if (c & 1) else ix)
                          ^ (y1 if (c & 2) else y0)
                          ^ (z1 if (c & 4) else z0)) & jnp.int32(T - 1)
                         ) + jnp.int32(lT2)
                    e = h + (h & jnp.int32(-128))
                    off = (l * 8 + c) * C
                    i0_v[pl.ds(off + g * 16, 16)] = e
                    i1_v[pl.ds(off + g * 16, 16)] = e + jnp.int32(128)
            return 0

        lax.fori_loop(0, C // 16, idx_g, 0, unroll=False)


def _acc_pass(x_v, f0_v, f1_v, out_v):
    for l in range(N_LEVELS):
        res = _RES[l]

        def acc_g(g, _, l=l, res=res):
            px = x_v[pl.ds(0 * C + g * 16, 16)] * jnp.float32(res)
            py = x_v[pl.ds(1 * C + g * 16, 16)] * jnp.float32(res)
            pz = x_v[pl.ds(2 * C + g * 16, 16)] * jnp.float32(res)
            wx = px - px.astype(jnp.int32).astype(jnp.float32)
            wy = py - py.astype(jnp.int32).astype(jnp.float32)
            wz = pz - pz.astype(jnp.int32).astype(jnp.float32)
            ox = jnp.float32(1.0) - wx
            oy = jnp.float32(1.0) - wy
            oz = jnp.float32(1.0) - wz
            acc0 = jnp.zeros((16,), jnp.float32)
            acc1 = jnp.zeros((16,), jnp.float32)
            for c in range(8):
                sx = wx if (c & 1) else ox
                sy = wy if (c & 2) else oy
                sz = wz if (c & 4) else oz
                wc = (sx * sy) * sz
                off = (l * 8 + c) * C + g * 16
                f0 = f0_v[pl.ds(off, 16)]
                f1 = f1_v[pl.ds(off, 16)]
                acc0 = acc0 + f0 * wc
                acc1 = acc1 + f1 * wc
            out_v[pl.ds((2 * l) * C + g * 16, 16)] = acc0
            out_v[pl.ds((2 * l + 1) * C + g * 16, 16)] = acc1
            return 0

        lax.fori_loop(0, C // 16, acc_g, 0, unroll=False)


def _tec_body(x_hbm, tbl_hbm, out_hbm,
              xa, xb, xia, xib, i0a, i1a, i0b, i1b,
              f0a, f1a, f0b, f1b, oa, ob,
              sga, sgb, soa, sob):
    wid = lax.axis_index("s") * NC + lax.axis_index("c")
    pbase = wid * P_PER_W
    iota = lax.iota(jnp.int32, 16)
    iota3 = iota * jnp.int32(3)

    x_refs = (xa, xb)
    xidx_refs = (xia, xib)
    idx_refs = ((i0a, i1a), (i0b, i1b))
    feat_refs = ((f0a, f1a), (f0b, f1b))
    out_refs = (oa, ob)
    gsems = (sga, sgb)
    osems = (soa, sob)

    def stage(par, ci):
        base = pbase + ci * C
        x_v = x_refs[par]
        xi_v = xidx_refs[par]
        i0, i1 = idx_refs[par]
        f0, f1 = feat_refs[par]
        sem = gsems[par]
        for d in range(3):
            def xg(g, _, d=d):
                xi_v[pl.ds(d * C + g * 16, 16)] = (
                    iota3 + jnp.int32(d) + 3 * (base + g * 16))
                return 0
            lax.fori_loop(0, C // 16, xg, 0, unroll=False)
        pltpu.make_async_copy(x_hbm.at[xi_v], x_v, sem).start()
        pltpu.make_async_copy(x_hbm.at[xi_v], x_v, sem).wait()
        _index_pass(x_v, i0, i1)
        pltpu.make_async_copy(tbl_hbm.at[i0], f0, sem).start()
        pltpu.make_async_copy(tbl_hbm.at[i1], f1, sem).start()

    def drain(par):
        i0, i1 = idx_refs[par]
        f0, f1 = feat_refs[par]
        sem = gsems[par]
        pltpu.make_async_copy(tbl_hbm.at[i0], f0, sem).wait()
        pltpu.make_async_copy(tbl_hbm.at[i1], f1, sem).wait()

    def out_start(par, ci):
        tc = (pbase // C) + ci
        o_v = out_refs[par]
        for j in range(NF):
            off = (j // 8) * (8 * N_POINTS) + tc * 1024 + (j % 8) * 128
            pltpu.make_async_copy(
                o_v.at[pl.ds(j * C, C)], out_hbm.at[pl.ds(off, C)], osems[par]
            ).start()

    def out_wait(par, ci):
        tc = (pbase // C) + ci
        o_v = out_refs[par]
        for j in range(NF):
            off = (j // 8) * (8 * N_POINTS) + tc * 1024 + (j % 8) * 128
            pltpu.make_async_copy(
                o_v.at[pl.ds(j * C, C)], out_hbm.at[pl.ds(off, C)], osems[par]
            ).wait()

    stage(0, 0)

    def chunk_pair(cp, _):
        for par in (0, 1):
            ci = cp * 2 + par

            @pl.when(ci + 1 < N_CHUNKS)
            def _():
                stage(1 - par, ci + 1)

            drain(par)

            @pl.when(ci >= 2)
            def _():
                out_wait(par, ci - 2)

            _acc_pass(x_refs[par], *feat_refs[par], out_refs[par])
            out_start(par, ci)
        return 0

    lax.fori_loop(0, N_CHUNKS // 2, chunk_pair, 0, unroll=False)

    for par in (0, 1):
        out_wait(par, N_CHUNKS - 2 + par)


@jax.jit
def kernel(x, table, mask):
    del mask
    x_flat = x.reshape(3 * N_POINTS)
    tbl = (table.reshape(N_LEVELS, T // 128, 128, F_PER)
           .transpose(0, 1, 3, 2)
           .reshape(N_LEVELS * T * F_PER))

    mesh = plsc.VectorSubcoreMesh(
        core_axis_name="c", subcore_axis_name="s", num_cores=NC, num_subcores=NS
    )
    f = pl.kernel(
        _tec_body,
        out_type=jax.ShapeDtypeStruct((N_POINTS * NF,), jnp.float32),
        mesh=mesh,
        scratch_types=[
            pltpu.VMEM((3 * C,), jnp.float32),
            pltpu.VMEM((3 * C,), jnp.float32),
            pltpu.VMEM((3 * C,), jnp.int32),
            pltpu.VMEM((3 * C,), jnp.int32),
            pltpu.VMEM((E_PER_F,), jnp.int32),
            pltpu.VMEM((E_PER_F,), jnp.int32),
            pltpu.VMEM((E_PER_F,), jnp.int32),
            pltpu.VMEM((E_PER_F,), jnp.int32),
            pltpu.VMEM((E_PER_F,), jnp.float32),
            pltpu.VMEM((E_PER_F,), jnp.float32),
            pltpu.VMEM((E_PER_F,), jnp.float32),
            pltpu.VMEM((E_PER_F,), jnp.float32),
            pltpu.VMEM((E_OUT,), jnp.float32),
            pltpu.VMEM((E_OUT,), jnp.float32),
            pltpu.SemaphoreType.DMA,
            pltpu.SemaphoreType.DMA,
            pltpu.SemaphoreType.DMA,
            pltpu.SemaphoreType.DMA,
        ],
    )
    flat = f(x_flat, tbl)
    return (flat.reshape(NF // 8, N_POINTS // 128, 8, 128)
            .transpose(1, 3, 0, 2)
            .reshape(N_POINTS, NF))


# speedup vs baseline: 2.6675x; 1.0015x over previous
import jax
import jax.numpy as jnp
import numpy as np
from jax import lax
from jax.experimental import pallas as pl
from jax.experimental.pallas import tpu as pltpu
from jax.experimental.pallas import tpu_sc as plsc

N_LEVELS = 12
F_PER = 2
LOG2_T = 19
T = 1 << LOG2_T
BASE_RES = 16
PER_LEVEL_SCALE = 2.0
N_POINTS = 524288

P1 = int(np.uint32(2654435761).view(np.int32))
P2 = int(np.uint32(805459861).view(np.int32))

NC, NS = 2, 16
NW = NC * NS
P_PER_W = N_POINTS // NW
C = 128
N_CHUNKS = P_PER_W // C
E_PER_F = N_LEVELS * 8 * C
NF = N_LEVELS * F_PER
E_OUT = NF * C

_RES = [int(np.ceil(BASE_RES * (PER_LEVEL_SCALE ** l))) for l in range(N_LEVELS)]
_DENSE = [(r + 1) ** 3 <= T for r in _RES]


def _index_pass(x_v, xb, i0_v, i1_v):
    for l in range(N_LEVELS):
        res = _RES[l]
        lT2 = l * T

        def idx_g(g, _, l=l, res=res, lT2=lT2):
            px = x_v[pl.ds(xb + 0 * C + g * 16, 16)] * jnp.float32(res)
            py = x_v[pl.ds(xb + 1 * C + g * 16, 16)] * jnp.float32(res)
            pz = x_v[pl.ds(xb + 2 * C + g * 16, 16)] * jnp.float32(res)
            ix = px.astype(jnp.int32)
            iy = py.astype(jnp.int32)
            iz = pz.astype(jnp.int32)
            if _DENSE[l]:
                s = res + 1
                y0 = iy * jnp.int32(s)
                y1 = y0 + jnp.int32(s)
                z0 = iz * jnp.int32(s * s)
                z1 = z0 + jnp.int32(s * s)
                x0 = ix + jnp.int32(lT2)
                x1 = x0 + jnp.int32(1)
                for c in range(8):
                    h = ((x1 if (c & 1) else x0)
                         + (y1 if (c & 2) else y0)
                         + (z1 if (c & 4) else z0))
                    e = h + (h & jnp.int32(-128))
                    off = (l * 8 + c) * C
                    i0_v[pl.ds(off + g * 16, 16)] = e
                    i1_v[pl.ds(off + g * 16, 16)] = e + jnp.int32(128)
            else:
                y0 = iy * jnp.int32(P1)
                y1 = y0 + jnp.int32(P1)
                z0 = iz * jnp.int32(P2)
                z1 = z0 + jnp.int32(P2)
                x1 = ix + jnp.int32(1)
                for c in range(8):
                    h = (((x1 if (c & 1) else ix)
                          ^ (y1 if (c & 2) else y0)
                          ^ (z1 if (c & 4) else z0)) & jnp.int32(T - 1)
                         ) + jnp.int32(lT2)
                    e = h + (h & jnp.int32(-128))
                    off = (l * 8 + c) * C
                    i0_v[pl.ds(off + g * 16, 16)] = e
                    i1_v[pl.ds(off + g * 16, 16)] = e + jnp.int32(128)
            return 0

        lax.fori_loop(0, C // 16, idx_g, 0, unroll=False)


def _acc_pass(x_v, xb, f0_v, f1_v, out_v):
    for l in range(N_LEVELS):
        res = _RES[l]

        def acc_g(g, _, l=l, res=res):
            px = x_v[pl.ds(xb + 0 * C + g * 16, 16)] * jnp.float32(res)
            py = x_v[pl.ds(xb + 1 * C + g * 16, 16)] * jnp.float32(res)
            pz = x_v[pl.ds(xb + 2 * C + g * 16, 16)] * jnp.float32(res)
            wx = px - px.astype(jnp.int32).astype(jnp.float32)
            wy = py - py.astype(jnp.int32).astype(jnp.float32)
            wz = pz - pz.astype(jnp.int32).astype(jnp.float32)
            ox = jnp.float32(1.0) - wx
            oy = jnp.float32(1.0) - wy
            oz = jnp.float32(1.0) - wz
            acc0 = jnp.zeros((16,), jnp.float32)
            acc1 = jnp.zeros((16,), jnp.float32)
            for c in range(8):
                sx = wx if (c & 1) else ox
                sy = wy if (c & 2) else oy
                sz = wz if (c & 4) else oz
                wc = (sx * sy) * sz
                off = (l * 8 + c) * C + g * 16
                f0 = f0_v[pl.ds(off, 16)]
                f1 = f1_v[pl.ds(off, 16)]
                acc0 = acc0 + f0 * wc
                acc1 = acc1 + f1 * wc
            out_v[pl.ds((2 * l) * C + g * 16, 16)] = acc0
            out_v[pl.ds((2 * l + 1) * C + g * 16, 16)] = acc1
            return 0

        lax.fori_loop(0, C // 16, acc_g, 0, unroll=False)


def _tec_body(x_hbm, tbl_hbm, out_hbm,
              x_v, xi0, xi1, i0a, i1a, i0b, i1b,
              f0a, f1a, f0b, f1b, oa, ob,
              sga, sgb, sxa, sxb, soa, sob):
    wid = lax.axis_index("s") * NC + lax.axis_index("c")
    pbase = wid * P_PER_W
    iota = lax.iota(jnp.int32, 16)
    iota3 = iota * jnp.int32(3)

    xidx_refs = (xi0, xi1)
    idx_refs = ((i0a, i1a), (i0b, i1b))
    feat_refs = ((f0a, f1a), (f0b, f1b))
    out_refs = (oa, ob)
    gsems = (sga, sgb)
    xsems = (sxa, sxb)
    osems = (soa, sob)
    XSZ = 3 * C

    def x_fire(par, ci):
        base = pbase + ci * C
        xi_v = xidx_refs[par]
        for d in range(3):
            def xg(g, _, d=d):
                xi_v[pl.ds(d * C + g * 16, 16)] = (
                    iota3 + jnp.int32(d) + 3 * (base + g * 16))
                return 0
            lax.fori_loop(0, C // 16, xg, 0, unroll=False)
        pltpu.make_async_copy(
            x_hbm.at[xi_v], x_v.at[pl.ds((ci % 4) * XSZ, XSZ)], xsems[par]
        ).start()

    def x_wait(par, ci):
        pltpu.make_async_copy(
            x_hbm.at[xidx_refs[par]], x_v.at[pl.ds((ci % 4) * XSZ, XSZ)],
            xsems[par],
        ).wait()

    def feat_fire(par, ci):
        xb = (ci % 4) * XSZ
        i0, i1 = idx_refs[par]
        f0, f1 = feat_refs[par]
        _index_pass(x_v, xb, i0, i1)
        pltpu.make_async_copy(tbl_hbm.at[i0], f0, gsems[par]).start()
        pltpu.make_async_copy(tbl_hbm.at[i1], f1, gsems[par]).start()

    def feat_drain(par):
        i0, i1 = idx_refs[par]
        f0, f1 = feat_refs[par]
        pltpu.make_async_copy(tbl_hbm.at[i0], f0, gsems[par]).wait()
        pltpu.make_async_copy(tbl_hbm.at[i1], f1, gsems[par]).wait()

    def out_start(par, ci):
        tc = (pbase // C) + ci
        o_v = out_refs[par]
        for j in range(NF):
            off = (j // 8) * (8 * N_POINTS) + tc * 1024 + (j % 8) * 128
            pltpu.make_async_copy(
                o_v.at[pl.ds(j * C, C)], out_hbm.at[pl.ds(off, C)], osems[par]
            ).start()

    def out_wait(par, ci):
        tc = (pbase // C) + ci
        o_v = out_refs[par]
        for j in range(NF):
            off = (j // 8) * (8 * N_POINTS) + tc * 1024 + (j % 8) * 128
            pltpu.make_async_copy(
                o_v.at[pl.ds(j * C, C)], out_hbm.at[pl.ds(off, C)], osems[par]
            ).wait()

    x_fire(0, 0)
    x_wait(0, 0)
    x_fire(1, 1)
    feat_fire(0, 0)

    def chunk_pair(cp, _):
        for par in (0, 1):
            ci = cp * 2 + par

            @pl.when(ci + 2 < N_CHUNKS)
            def _():
                x_fire(par, ci + 2)

            @pl.when(ci + 1 < N_CHUNKS)
            def _():
                x_wait(1 - par, ci + 1)
                feat_fire(1 - par, ci + 1)

            feat_drain(par)

            @pl.when(ci >= 2)
            def _():
                out_wait(par, ci - 2)

            _acc_pass(x_v, (ci % 4) * XSZ, *feat_refs[par], out_refs[par])
            out_start(par, ci)
        return 0

    lax.fori_loop(0, N_CHUNKS // 2, chunk_pair, 0, unroll=False)

    for par in (0, 1):
        out_wait(par, N_CHUNKS - 2 + par)


@jax.jit
def kernel(x, table, mask):
    del mask
    x_flat = x.reshape(3 * N_POINTS)
    tbl = (table.reshape(N_LEVELS, T // 128, 128, F_PER)
           .transpose(0, 1, 3, 2)
           .reshape(N_LEVELS * T * F_PER))

    mesh = plsc.VectorSubcoreMesh(
        core_axis_name="c", subcore_axis_name="s", num_cores=NC, num_subcores=NS
    )
    f = pl.kernel(
        _tec_body,
        out_type=jax.ShapeDtypeStruct((N_POINTS * NF,), jnp.float32),
        mesh=mesh,
        scratch_types=[
            pltpu.VMEM((4 * 3 * C,), jnp.float32),
            pltpu.VMEM((3 * C,), jnp.int32),
            pltpu.VMEM((3 * C,), jnp.int32),
            pltpu.VMEM((E_PER_F,), jnp.int32),
            pltpu.VMEM((E_PER_F,), jnp.int32),
            pltpu.VMEM((E_PER_F,), jnp.int32),
            pltpu.VMEM((E_PER_F,), jnp.int32),
            pltpu.VMEM((E_PER_F,), jnp.float32),
            pltpu.VMEM((E_PER_F,), jnp.float32),
            pltpu.VMEM((E_PER_F,), jnp.float32),
            pltpu.VMEM((E_PER_F,), jnp.float32),
            pltpu.VMEM((E_OUT,), jnp.float32),
            pltpu.VMEM((E_OUT,), jnp.float32),
            pltpu.SemaphoreType.DMA,
            pltpu.SemaphoreType.DMA,
            pltpu.SemaphoreType.DMA,
            pltpu.SemaphoreType.DMA,
            pltpu.SemaphoreType.DMA,
            pltpu.SemaphoreType.DMA,
        ],
    )
    flat = f(x_flat, tbl)
    return (flat.reshape(NF // 8, N_POINTS // 128, 8, 128)
            .transpose(1, 3, 0, 2)
            .reshape(N_POINTS, NF))
